# baseline (speedup 1.0000x reference)
import functools

import jax
import jax.numpy as jnp
from jax import lax
from jax.experimental import pallas as pl
from jax.experimental.pallas import tpu as pltpu
from jax.experimental.pallas import tpu_sc as plsc

N = 10000
E = 320000
D = 128
DH = D // 2
NPAD = 10240
NMR = 10016
NC, NS, L = 2, 16, 16
CHUNK = 128
CH_PER_T = 162
ETOT = NS * CH_PER_T * CHUNK
NMASKP = 1504
ROWS_PER_S = NPAD // NS

_f32 = jnp.float32
_i32 = jnp.int32


def _enc_body(pw_ref, x_ref, wt_ref, o_ref):
    x = x_ref[...]
    a = pw_ref[...]
    px = jnp.where(x >= 0.0, x, x * a)
    h = jnp.dot(px, wt_ref[...], preferred_element_type=_f32)
    o_ref[...] = jnp.stack([h[:, :DH], h[:, DH:]], axis=0)


def _encoder(xp, pw_row, w_enc_t):
    return pl.pallas_call(
        _enc_body,
        grid=(NPAD // 256,),
        in_specs=[
            pl.BlockSpec((1, D), lambda i: (0, 0)),
            pl.BlockSpec((256, D), lambda i: (i, 0)),
            pl.BlockSpec((D, D), lambda i: (0, 0)),
        ],
        out_specs=pl.BlockSpec((NC, 256, DH), lambda i: (0, i, 0)),
        out_shape=jax.ShapeDtypeStruct((NC, NPAD, DH), _f32),
    )(pw_row, xp, w_enc_t)


def _sc_body(htab, epack, mrinit, mskp, outh, outc,
             mr, mb, pbig, gb0, gb1, db0, db1, rb0, rb1, ob0, ob1,
             acc, cacc, semg0, semg1):
    cid = lax.axis_index("c")
    sid = lax.axis_index("s")

    z16 = jnp.zeros((L,), _f32)
    ones16 = jnp.full((L,), 1.0, _f32)
    splat_n = jnp.full((L,), N, _i32)
    lane = lax.iota(_i32, L)
    cidoff = cid * NPAD

    pltpu.sync_copy(mrinit, mr)
    pltpu.sync_copy(mskp, mb)
    pltpu.sync_copy(epack.at[sid], pbig.at[pl.ds(0, CH_PER_T)])

    for i in range(CHUNK // L):
        pbig[CH_PER_T, pl.ds(i * L, L)] = splat_n

    def mask_loop(i, c):
        mi = mb[pl.ds(i * L, L)]
        plsc.store_scatter(mr, [mi], splat_n)
        return c

    lax.fori_loop(0, NMASKP // L, mask_loop, 0)

    def zero_loop(r, c):
        for c8 in range(DH // L):
            rb0[r, pl.ds(c8 * L, L)] = z16
        ob0[r, pl.ds(0, L)] = z16
        ob1[r, pl.ds(0, L)] = z16
        return c

    lax.fori_loop(0, CHUNK, zero_loop, 0)

    row0 = sid * ROWS_PER_S
    for j in range(ROWS_PER_S // CHUNK):
        pltpu.sync_copy(rb0, acc.at[pl.ds(row0 + j * CHUNK, CHUNK)])

    @pl.when(cid == 0)
    def _():
        for j in range(ROWS_PER_S // CHUNK):
            pltpu.sync_copy(ob0, cacc.at[pl.ds(row0 + j * CHUNK, CHUNK)])

    plsc.subcore_barrier()

    def build(row, gb, db):
        for i in range(CHUNK // L):
            w = pbig[row, pl.ds(i * L, L)]
            sv = w & 0x3FFF
            db[pl.ds(i * L, L)] = (w >> 14) & 0x3FFF
            gb[pl.ds(i * L, L)] = plsc.load_gather(mr, [sv]) + cidoff

    def do_counts(row, ob, db):
        @pl.when(cid == 0)
        def _():
            for i in range(CHUNK // L):
                cv = (pbig[row, pl.ds(i * L, L)] >> 28) & 0xF
                plsc.store_scatter(ob, [lane + (i * L), cv], ones16)
            pltpu.sync_copy(ob, cacc.at[db], add=True)
            for i in range(CHUNK // L):
                cv = (pbig[row, pl.ds(i * L, L)] >> 28) & 0xF
                plsc.store_scatter(ob, [lane + (i * L), cv], z16)

    build(0, gb0, db0)
    pltpu.async_copy(htab.at[gb0], rb0, semg0)

    def chunk_loop(k, c):
        e0 = 2 * k
        build(e0 + 1, gb1, db1)
        pltpu.async_copy(htab.at[gb1], rb1, semg1)
        pltpu.make_async_copy(htab.at[gb0], rb0, semg0).wait()
        pltpu.sync_copy(rb0, acc.at[db0], add=True)
        do_counts(e0, ob0, db0)
        build(e0 + 2, gb0, db0)
        pltpu.async_copy(htab.at[gb0], rb0, semg0)
        pltpu.make_async_copy(htab.at[gb1], rb1, semg1).wait()
        pltpu.sync_copy(rb1, acc.at[db1], add=True)
        do_counts(e0 + 1, ob1, db1)
        return c

    lax.fori_loop(0, CH_PER_T // 2, chunk_loop, 0)
    pltpu.make_async_copy(htab.at[gb0], rb0, semg0).wait()
    plsc.subcore_barrier()

    for j in range(ROWS_PER_S // CHUNK):
        r0 = row0 + j * CHUNK
        pltpu.sync_copy(acc.at[pl.ds(r0, CHUNK)], outh.at[cid, pl.ds(r0, CHUNK)])

    @pl.when(cid == 0)
    def _():
        for j in range(ROWS_PER_S // CHUNK):
            r0 = row0 + j * CHUNK
            pltpu.sync_copy(cacc.at[pl.ds(r0, CHUNK)], outc.at[pl.ds(r0, CHUNK)])


_sc_main = functools.partial(
    pl.kernel,
    out_type=[
        jax.ShapeDtypeStruct((NC, NPAD, DH), _f32),
        jax.ShapeDtypeStruct((NPAD, L), _f32),
    ],
    mesh=plsc.VectorSubcoreMesh(core_axis_name="c", subcore_axis_name="s"),
    compiler_params=pltpu.CompilerParams(
        needs_layout_passes=False, use_tc_tiling_on_sc=False),
    scratch_types=[
        pltpu.VMEM((NMR,), _i32),
        pltpu.VMEM((NMASKP,), _i32),
        pltpu.VMEM((CH_PER_T + 1, CHUNK), _i32),
        pltpu.VMEM((CHUNK,), _i32),
        pltpu.VMEM((CHUNK,), _i32),
        pltpu.VMEM((CHUNK,), _i32),
        pltpu.VMEM((CHUNK,), _i32),
        pltpu.VMEM((CHUNK, DH), _f32),
        pltpu.VMEM((CHUNK, DH), _f32),
        pltpu.VMEM((CHUNK, L), _f32),
        pltpu.VMEM((CHUNK, L), _f32),
        pltpu.VMEM_SHARED((NPAD, DH), _f32),
        pltpu.VMEM_SHARED((NPAD, L), _f32),
        pltpu.SemaphoreType.DMA,
        pltpu.SemaphoreType.DMA,
    ],
)(_sc_body)


def _mlp_body(ph_ref, pc_ref, t_ref, cst_ref, w1t_ref, b1_ref, w2t_ref,
              b2_ref, o_ref):
    p = ph_ref[...]
    c = pc_ref[...]
    a = (jnp.concatenate([p[0], p[1]], axis=-1) + cst_ref[...]
         + jnp.dot(c, t_ref[...], preferred_element_type=_f32))
    h1 = jnp.maximum(jnp.dot(a, w1t_ref[...], preferred_element_type=_f32)
                     + b1_ref[...], 0.0)
    o_ref[...] = jnp.dot(h1, w2t_ref[...], preferred_element_type=_f32) + b2_ref[...]


def _mlp(outh, outc, tc16, cst_row, w1t, b1r, w2t, b2r):
    return pl.pallas_call(
        _mlp_body,
        grid=(NPAD // 256,),
        in_specs=[
            pl.BlockSpec((NC, 256, DH), lambda i: (0, i, 0)),
            pl.BlockSpec((256, L), lambda i: (i, 0)),
            pl.BlockSpec((L, D), lambda i: (0, 0)),
            pl.BlockSpec((1, D), lambda i: (0, 0)),
            pl.BlockSpec((D, 2 * D), lambda i: (0, 0)),
            pl.BlockSpec((1, 2 * D), lambda i: (0, 0)),
            pl.BlockSpec((2 * D, D), lambda i: (0, 0)),
            pl.BlockSpec((1, D), lambda i: (0, 0)),
        ],
        out_specs=pl.BlockSpec((256, D), lambda i: (i, 0)),
        out_shape=jax.ShapeDtypeStruct((NPAD, D), _f32),
    )(outh, outc, tc16, cst_row, w1t, b1r, w2t, b2r)


def kernel(x, edge_index, edge_attr, masked_node_indices, prelu_w, W_enc,
           emb1, emb2, W1, b1, W2, b2):
    xp = jnp.zeros((NPAD, D), _f32).at[:N].set(x)
    pw_row = jnp.broadcast_to(prelu_w.astype(_f32), (1, D))
    w_enc_t = W_enc.T

    loops = jnp.arange(N, dtype=_i32)
    npad_e = ETOT - E - N
    srcf = jnp.concatenate([edge_index[0], loops,
                            jnp.full((npad_e,), N, _i32)])
    dstf = jnp.concatenate([edge_index[1], loops,
                            jnp.full((npad_e,), N, _i32)])
    combo = edge_attr[:, 0] * 3 + edge_attr[:, 1]
    cmbf = jnp.concatenate([combo.astype(_i32),
                            jnp.full((N + npad_e,), 15, _i32)])
    epack = srcf | (dstf << 14) | (cmbf << 28)
    mrinit = jnp.minimum(jnp.arange(NMR, dtype=_i32), N)
    mskp = jnp.concatenate([masked_node_indices.astype(_i32),
                            masked_node_indices[:NMASKP - 1500].astype(_i32)])

    tc9 = jnp.repeat(emb1[:3], 3, axis=0) + jnp.tile(emb2[:3], (3, 1))
    tc16 = jnp.zeros((L, D), _f32).at[:9].set(tc9)
    cst_row = (emb1[4] + emb2[0]).reshape(1, D)
    w1t, w2t = W1.T, W2.T
    b1r, b2r = b1.reshape(1, 2 * D), b2.reshape(1, D)

    htab2 = _encoder(xp, pw_row, w_enc_t)
    htabf = htab2.reshape(NC * NPAD, DH)
    epack3 = epack.reshape(NS, CH_PER_T, CHUNK)
    outh, outc = _sc_main(htabf, epack3, mrinit, mskp)
    out_full = _mlp(outh, outc, tc16, cst_row, w1t, b1r, w2t, b2r)
    return out_full[:N]


# speedup vs baseline: 3.4454x; 3.4454x over previous
import functools

import jax
import jax.numpy as jnp
from jax import lax
from jax.experimental import pallas as pl
from jax.experimental.pallas import tpu as pltpu
from jax.experimental.pallas import tpu_sc as plsc

N = 10000
E = 320000
D = 128
DH = D // 2
NPAD = 10240
NACC = 10016
NC, NS, L = 2, 16, 16
CHUNK = 128
CH_PER_T = 162
CH_STAGE = 165
ETOT = NS * CH_STAGE * CHUNK
NMASKP = 1536
ROWS_PER_S = NACC // NS

_f32 = jnp.float32
_i32 = jnp.int32


def _enc_body(pw_ref, x_ref, wt_ref, o_ref):
    x = x_ref[...]
    a = pw_ref[...]
    px = jnp.where(x >= 0.0, x, x * a)
    h = jnp.dot(px, wt_ref[...], preferred_element_type=_f32)
    o_ref[...] = jnp.stack([h[:, :DH], h[:, DH:]], axis=0)


def _encoder(xp, pw_row, w_enc_t):
    return pl.pallas_call(
        _enc_body,
        grid=(NPAD // 256,),
        in_specs=[
            pl.BlockSpec((1, D), lambda i: (0, 0)),
            pl.BlockSpec((256, D), lambda i: (i, 0)),
            pl.BlockSpec((D, D), lambda i: (0, 0)),
        ],
        out_specs=pl.BlockSpec((NC, 256, DH), lambda i: (0, i, 0)),
        out_shape=jax.ShapeDtypeStruct((NC, NPAD, DH), _f32),
    )(pw_row, xp, w_enc_t)


def _sc_body(htab, epack, mskp, outh, outc,
             mbs, ib0, ib1, gb0, gb1, db0, db1, cb0, cb1, rb0, rb1, ob0,
             spm_h, acc, cacc, semg0, semg1, semi0, semi1):
    cid = lax.axis_index("c")
    sid = lax.axis_index("s")

    z16 = jnp.zeros((L,), _f32)
    ones16 = jnp.full((L,), 1.0, _f32)
    lane = lax.iota(_i32, L)

    row0 = sid * ROWS_PER_S
    pltpu.sync_copy(htab.at[cid, pl.ds(row0, ROWS_PER_S)],
                    spm_h.at[pl.ds(row0, ROWS_PER_S)])
    pltpu.sync_copy(mskp.at[sid], mbs)

    def zero_loop(r, c):
        for c8 in range(DH // L):
            rb0[r, pl.ds(c8 * L, L)] = z16
        ob0[r, pl.ds(0, L)] = z16
        return c

    lax.fori_loop(0, CHUNK, zero_loop, 0)

    for j in range(ROWS_PER_S // CHUNK):
        pltpu.sync_copy(rb0, acc.at[pl.ds(row0 + j * CHUNK, CHUNK)])
    pltpu.sync_copy(rb0.at[pl.ds(0, ROWS_PER_S % CHUNK)],
                    acc.at[pl.ds(row0 + (ROWS_PER_S // CHUNK) * CHUNK,
                                 ROWS_PER_S % CHUNK)])

    @pl.when(cid == 0)
    def _():
        for j in range(ROWS_PER_S // CHUNK):
            pltpu.sync_copy(ob0, cacc.at[pl.ds(row0 + j * CHUNK, CHUNK)])
        pltpu.sync_copy(ob0.at[pl.ds(0, ROWS_PER_S % CHUNK)],
                        cacc.at[pl.ds(row0 + (ROWS_PER_S // CHUNK) * CHUNK,
                                      ROWS_PER_S % CHUNK)])

    plsc.subcore_barrier()

    pltpu.sync_copy(rb0.at[pl.ds(0, NMASKP // NS)], spm_h.at[mbs])
    plsc.subcore_barrier()

    def build(ib, gb, db, cb):
        for i in range(CHUNK // L):
            w = ib[pl.ds(i * L, L)]
            gb[pl.ds(i * L, L)] = w & 0x3FFF
            db[pl.ds(i * L, L)] = (w >> 14) & 0x3FFF
            cb[pl.ds(i * L, L)] = (w >> 28) & 0xF

    def do_counts(cb, db):
        @pl.when(cid == 0)
        def _():
            for i in range(CHUNK // L):
                cv = cb[pl.ds(i * L, L)]
                plsc.store_scatter(ob0, [lane + (i * L), cv], ones16)
            pltpu.sync_copy(ob0, cacc.at[db], add=True)
            for i in range(CHUNK // L):
                cv = cb[pl.ds(i * L, L)]
                plsc.store_scatter(ob0, [lane + (i * L), cv], z16)

    pltpu.sync_copy(epack.at[sid, 0], ib0)
    build(ib0, gb0, db0, cb0)
    pltpu.async_copy(epack.at[sid, 1], ib1, semi1)
    pltpu.async_copy(epack.at[sid, 2], ib0, semi0)
    pltpu.async_copy(spm_h.at[gb0], rb0, semg0)

    def chunk_loop(k, c):
        e0 = 2 * k
        pltpu.make_async_copy(epack.at[sid, e0 + 1], ib1, semi1).wait()
        build(ib1, gb1, db1, cb1)
        pltpu.async_copy(spm_h.at[gb1], rb1, semg1)
        pltpu.async_copy(epack.at[sid, e0 + 3], ib1, semi1)
        pltpu.make_async_copy(spm_h.at[gb0], rb0, semg0).wait()
        pltpu.sync_copy(rb0, acc.at[db0], add=True)
        do_counts(cb0, db0)
        pltpu.make_async_copy(epack.at[sid, e0 + 2], ib0, semi0).wait()
        build(ib0, gb0, db0, cb0)
        pltpu.async_copy(spm_h.at[gb0], rb0, semg0)
        pltpu.async_copy(epack.at[sid, e0 + 4], ib0, semi0)
        pltpu.make_async_copy(spm_h.at[gb1], rb1, semg1).wait()
        pltpu.sync_copy(rb1, acc.at[db1], add=True)
        do_counts(cb1, db1)
        return c

    lax.fori_loop(0, CH_PER_T // 2, chunk_loop, 0)
    pltpu.make_async_copy(spm_h.at[gb0], rb0, semg0).wait()
    pltpu.make_async_copy(epack.at[sid, CH_PER_T + 1], ib1, semi1).wait()
    pltpu.make_async_copy(epack.at[sid, CH_PER_T + 2], ib0, semi0).wait()
    plsc.subcore_barrier()

    pltpu.sync_copy(acc.at[pl.ds(row0, ROWS_PER_S)],
                    outh.at[cid, pl.ds(row0, ROWS_PER_S)])

    @pl.when(cid == 0)
    def _():
        pltpu.sync_copy(cacc.at[pl.ds(row0, ROWS_PER_S)],
                        outc.at[pl.ds(row0, ROWS_PER_S)])


_sc_main = functools.partial(
    pl.kernel,
    out_type=[
        jax.ShapeDtypeStruct((NC, NPAD, DH), _f32),
        jax.ShapeDtypeStruct((NPAD, L), _f32),
    ],
    mesh=plsc.VectorSubcoreMesh(core_axis_name="c", subcore_axis_name="s"),
    compiler_params=pltpu.CompilerParams(
        needs_layout_passes=False, use_tc_tiling_on_sc=False),
    scratch_types=[
        pltpu.VMEM((NMASKP // NS,), _i32),
        pltpu.VMEM((CHUNK,), _i32),
        pltpu.VMEM((CHUNK,), _i32),
        pltpu.VMEM((CHUNK,), _i32),
        pltpu.VMEM((CHUNK,), _i32),
        pltpu.VMEM((CHUNK,), _i32),
        pltpu.VMEM((CHUNK,), _i32),
        pltpu.VMEM((CHUNK,), _i32),
        pltpu.VMEM((CHUNK,), _i32),
        pltpu.VMEM((CHUNK, DH), _f32),
        pltpu.VMEM((CHUNK, DH), _f32),
        pltpu.VMEM((CHUNK, L), _f32),
        pltpu.VMEM_SHARED((NACC, DH), _f32),
        pltpu.VMEM_SHARED((NACC, DH), _f32),
        pltpu.VMEM_SHARED((NACC, L), _f32),
        pltpu.SemaphoreType.DMA,
        pltpu.SemaphoreType.DMA,
        pltpu.SemaphoreType.DMA,
        pltpu.SemaphoreType.DMA,
    ],
)(_sc_body)


def _mlp_body(ph_ref, pc_ref, t_ref, cst_ref, w1t_ref, b1_ref, w2t_ref,
              b2_ref, o_ref):
    p = ph_ref[...]
    c = pc_ref[...]
    a = (jnp.concatenate([p[0], p[1]], axis=-1) + cst_ref[...]
         + jnp.dot(c, t_ref[...], preferred_element_type=_f32))
    h1 = jnp.maximum(jnp.dot(a, w1t_ref[...], preferred_element_type=_f32)
                     + b1_ref[...], 0.0)
    o_ref[...] = jnp.dot(h1, w2t_ref[...], preferred_element_type=_f32) + b2_ref[...]


def _mlp(outh, outc, tc16, cst_row, w1t, b1r, w2t, b2r):
    return pl.pallas_call(
        _mlp_body,
        grid=(NPAD // 256,),
        in_specs=[
            pl.BlockSpec((NC, 256, DH), lambda i: (0, i, 0)),
            pl.BlockSpec((256, L), lambda i: (i, 0)),
            pl.BlockSpec((L, D), lambda i: (0, 0)),
            pl.BlockSpec((1, D), lambda i: (0, 0)),
            pl.BlockSpec((D, 2 * D), lambda i: (0, 0)),
            pl.BlockSpec((1, 2 * D), lambda i: (0, 0)),
            pl.BlockSpec((2 * D, D), lambda i: (0, 0)),
            pl.BlockSpec((1, D), lambda i: (0, 0)),
        ],
        out_specs=pl.BlockSpec((256, D), lambda i: (i, 0)),
        out_shape=jax.ShapeDtypeStruct((NPAD, D), _f32),
    )(outh, outc, tc16, cst_row, w1t, b1r, w2t, b2r)


def kernel(x, edge_index, edge_attr, masked_node_indices, prelu_w, W_enc,
           emb1, emb2, W1, b1, W2, b2):
    xp = jnp.zeros((NPAD, D), _f32).at[:N].set(x)
    pw_row = jnp.broadcast_to(prelu_w.astype(_f32), (1, D))
    w_enc_t = W_enc.T

    loops = jnp.arange(N, dtype=_i32)
    nproc = NS * CH_PER_T * CHUNK
    npad_e = nproc - E - N
    srcf = jnp.concatenate([edge_index[0], loops,
                            jnp.full((npad_e,), N, _i32)])
    dstf = jnp.concatenate([edge_index[1], loops,
                            jnp.full((npad_e,), N, _i32)])
    combo = edge_attr[:, 0] * 3 + edge_attr[:, 1]
    cmbf = jnp.concatenate([combo.astype(_i32),
                            jnp.full((N + npad_e,), 15, _i32)])
    epack = srcf | (dstf << 14) | (cmbf << 28)
    mskp = jnp.concatenate([masked_node_indices.astype(_i32),
                            masked_node_indices[:NMASKP - 1500].astype(_i32)])
    mskp2 = mskp.reshape(NS, NMASKP // NS)

    tc9 = jnp.repeat(emb1[:3], 3, axis=0) + jnp.tile(emb2[:3], (3, 1))
    tc16 = jnp.zeros((L, D), _f32).at[:9].set(tc9)
    cst_row = (emb1[4] + emb2[0]).reshape(1, D)
    w1t, w2t = W1.T, W2.T
    b1r, b2r = b1.reshape(1, 2 * D), b2.reshape(1, D)

    htab2 = _encoder(xp, pw_row, w_enc_t)
    pw = jnp.int32(N) | (jnp.int32(N) << 14) | (jnp.int32(15) << 28)
    epack3 = jnp.concatenate(
        [epack.reshape(NS, CH_PER_T, CHUNK),
         jnp.full((NS, CH_STAGE - CH_PER_T, CHUNK), pw, _i32)], axis=1)
    outh, outc = _sc_main(htab2, epack3, mskp2)
    out_full = _mlp(outh, outc, tc16, cst_row, w1t, b1r, w2t, b2r)
    return out_full[:N]
